# Initial kernel scaffold; baseline (speedup 1.0000x reference)
import jax
import jax.numpy as jnp
from jax.experimental import pallas as pl


def kernel(x, edge_index, lin_w, lin_b, conv_w, conv_b):
    raise NotImplementedError("write your pallas kernel here")


# speedup vs baseline: 13.2766x; 13.2766x over previous
import functools

import jax
import jax.numpy as jnp
from jax import lax
from jax.experimental import pallas as pl
from jax.experimental.pallas import tpu as pltpu
from jax.experimental.pallas import tpu_sc as plsc

NC = 2
NS = 16
NW = NC * NS
LANES = 16
K_E = 80


def _mm_body(x_ref, w_ref, o_ref):
    o_ref[...] = jnp.dot(x_ref[...], w_ref[...],
                         preferred_element_type=jnp.float32)


def _tc_matmul(x_pad, w_pad, blk):
    np_, d = x_pad.shape
    cols = w_pad.shape[1]
    return pl.pallas_call(
        _mm_body,
        grid=(np_ // blk,),
        in_specs=[
            pl.BlockSpec((blk, d), lambda i: (i, 0)),
            pl.BlockSpec((d, cols), lambda i: (0, 0)),
        ],
        out_specs=pl.BlockSpec((blk, cols), lambda i: (i, 0)),
        out_shape=jax.ShapeDtypeStruct((np_, cols), jnp.float32),
    )(x_pad, w_pad)


def _y_body(degp_ref, xw_ref, y_ref):
    deg = 1.0 + jnp.sum(degp_ref[...], axis=0)
    dinv = lax.rsqrt(deg)
    y_ref[...] = xw_ref[:, :128] * dinv[:, None]


def _tc_scale(degp, xw256, blk):
    np_ = xw256.shape[0]
    return pl.pallas_call(
        _y_body,
        grid=(np_ // blk,),
        in_specs=[
            pl.BlockSpec((NW, blk), lambda i: (0, i)),
            pl.BlockSpec((blk, 256), lambda i: (i, 0)),
        ],
        out_specs=pl.BlockSpec((blk, 128), lambda i: (i, 0)),
        out_shape=jax.ShapeDtypeStruct((np_, 128), jnp.float32),
    )(degp, xw256)


def _out_body(degp_ref, p_ref, xw_ref, cb_ref, o_ref):
    deg = 1.0 + jnp.sum(degp_ref[...], axis=0)
    dinv = lax.rsqrt(deg)[:, None]
    ps = p_ref[0] + p_ref[1]
    o_ref[...] = dinv * (ps + dinv * xw_ref[:, :128]) + cb_ref[0:1, :]


def _tc_combine(degp, parts, xw256, convb, blk):
    np_ = xw256.shape[0]
    return pl.pallas_call(
        _out_body,
        grid=(np_ // blk,),
        in_specs=[
            pl.BlockSpec((NW, blk), lambda i: (0, i)),
            pl.BlockSpec((2, blk, 128), lambda i: (0, i, 0)),
            pl.BlockSpec((blk, 256), lambda i: (i, 0)),
            pl.BlockSpec((8, 128), lambda i: (0, 0)),
        ],
        out_specs=pl.BlockSpec((blk, 128), lambda i: (i, 0)),
        out_shape=jax.ShapeDtypeStruct((np_, 128), jnp.float32),
    )(degp, parts, xw256, convb)


def _attn_deg_kernel(a, b, row, col, np_, e):
    ept = e // NW
    mesh = plsc.VectorSubcoreMesh(core_axis_name="c", subcore_axis_name="s")

    @functools.partial(
        pl.kernel,
        out_type=[
            jax.ShapeDtypeStruct((e,), jnp.float32),
            jax.ShapeDtypeStruct((NW, np_), jnp.float32),
        ],
        mesh=mesh,
        compiler_params=pltpu.CompilerParams(needs_layout_passes=False),
        scratch_types=[
            pltpu.VMEM((K_E,), jnp.int32),
            pltpu.VMEM((K_E,), jnp.int32),
            pltpu.VMEM((K_E,), jnp.float32),
            pltpu.VMEM((K_E,), jnp.float32),
            pltpu.VMEM((K_E,), jnp.float32),
            pltpu.VMEM((np_,), jnp.float32),
            pltpu.SemaphoreType.DMA,
            pltpu.SemaphoreType.DMA,
        ],
    )
    def k2(a_hbm, b_hbm, row_hbm, col_hbm, attn_hbm, degp_hbm,
           ridx, cidx, av, bv, at, degv, sem1, sem2):
        wid = lax.axis_index("s") * NC + lax.axis_index("c")
        base = wid * ept

        def zero(i, _):
            degv[pl.ds(i * LANES, LANES)] = jnp.zeros((LANES,), jnp.float32)
            return 0
        lax.fori_loop(0, np_ // LANES, zero, 0)

        def chunk(ci, _):
            off = base + ci * K_E
            pltpu.sync_copy(row_hbm.at[pl.ds(off, K_E)], ridx)
            pltpu.sync_copy(col_hbm.at[pl.ds(off, K_E)], cidx)
            cp1 = pltpu.async_copy(a_hbm.at[ridx], av, sem1)
            cp2 = pltpu.async_copy(b_hbm.at[cidx], bv, sem2)
            cp1.wait()
            cp2.wait()

            def inner(m, _):
                s = pl.ds(m * LANES, LANES)
                z = av[s] + bv[s]
                sig = 1.0 / (1.0 + jnp.exp(-z))
                at[s] = sig
                plsc.addupdate_scatter(degv, [cidx[s]], sig)
                return 0
            lax.fori_loop(0, K_E // LANES, inner, 0)
            pltpu.sync_copy(at, attn_hbm.at[pl.ds(off, K_E)])
            return 0
        lax.fori_loop(0, ept // K_E, chunk, 0)
        pltpu.sync_copy(degv, degp_hbm.at[wid])

    return k2(a, b, row, col)


def _gather_scatter_kernel(y, row, col, attn, np_, e):
    ept = e // NW
    rows_per_tile = np_ // NS
    mesh = plsc.VectorSubcoreMesh(core_axis_name="c", subcore_axis_name="s")

    @functools.partial(
        pl.kernel,
        out_type=jax.ShapeDtypeStruct((NC, np_, 128), jnp.float32),
        mesh=mesh,
        compiler_params=pltpu.CompilerParams(needs_layout_passes=False),
        scratch_types=[
            pltpu.VMEM((K_E,), jnp.int32),
            pltpu.VMEM((K_E,), jnp.int32),
            pltpu.VMEM((K_E,), jnp.float32),
            pltpu.VMEM((K_E, 128), jnp.float32),
            pltpu.VMEM_SHARED((np_, 128), jnp.float32),
            pltpu.SemaphoreType.DMA,
        ],
    )
    def k4(y_hbm, row_hbm, col_hbm, attn_hbm, parts_hbm,
           ridx, cidx, atb, rows, acc, gsem):
        cid = lax.axis_index("c")
        sid = lax.axis_index("s")
        wid = sid * NC + cid
        base = wid * ept

        def zrow(i, _):
            for j in range(8):
                rows[i, pl.ds(j * LANES, LANES)] = jnp.zeros((LANES,),
                                                             jnp.float32)
            return 0
        lax.fori_loop(0, K_E, zrow, 0)
        r0 = sid * rows_per_tile
        for j in range(rows_per_tile // K_E):
            pltpu.sync_copy(rows, acc.at[pl.ds(r0 + j * K_E, K_E)])
        plsc.subcore_barrier()

        def chunk(ci, _):
            off = base + ci * K_E
            pltpu.sync_copy(row_hbm.at[pl.ds(off, K_E)], ridx)
            pltpu.sync_copy(col_hbm.at[pl.ds(off, K_E)], cidx)
            pltpu.sync_copy(attn_hbm.at[pl.ds(off, K_E)], atb)
            pltpu.async_copy(y_hbm.at[ridx], rows, gsem).wait()

            def scale(m, _):
                av16 = atb[pl.ds(m * LANES, LANES)]
                for t in range(LANES):
                    ei = m * LANES + t
                    s = av16[t]
                    for j in range(8):
                        d = pl.ds(j * LANES, LANES)
                        rows[ei, d] = rows[ei, d] * s
                return 0
            lax.fori_loop(0, K_E // LANES, scale, 0)
            pltpu.sync_copy(rows, acc.at[cidx], add=True)
            return 0
        lax.fori_loop(0, ept // K_E, chunk, 0)

        plsc.subcore_barrier()
        for j in range(rows_per_tile // K_E):
            rr = r0 + j * K_E
            pltpu.sync_copy(acc.at[pl.ds(rr, K_E)], rows)
            pltpu.sync_copy(rows, parts_hbm.at[cid, pl.ds(rr, K_E)])

    return k4(y, row, col, attn)


def kernel(x, edge_index, lin_w, lin_b, conv_w, conv_b):
    n, d = x.shape
    e = edge_index.shape[1]
    blk = 1280
    np_ = ((n + blk - 1) // blk) * blk

    x_pad = jnp.zeros((np_, d), jnp.float32).at[:n].set(x)
    w_pad = (jnp.zeros((d, 256), jnp.float32)
             .at[:, :128].set(conv_w.T)
             .at[:, 128].set(lin_w[0, :d])
             .at[:, 129].set(lin_w[0, d:]))

    xw256 = _tc_matmul(x_pad, w_pad, blk)
    a = xw256[:, 128] + lin_b[0]
    b = xw256[:, 129]
    row = edge_index[0]
    col = edge_index[1]

    attn, degp = _attn_deg_kernel(a, b, row, col, np_, e)
    y = _tc_scale(degp, xw256, blk)
    parts = _gather_scatter_kernel(y, row, col, attn, np_, e)

    convb = jnp.broadcast_to(conv_b[None, :], (8, 128))
    res = _tc_combine(degp, parts, xw256, convb, blk)
    return res[:n]
